# baseline (speedup 1.0000x reference)
import functools

import jax
import jax.numpy as jnp
from jax import lax
from jax.experimental import pallas as pl
from jax.experimental.pallas import tpu as pltpu
from jax.experimental.pallas import tpu_sc as plsc

E = 1000000
R = 1000
B = 16384
D = 64
RW = 128
NC = 2
NS = 16
L = 16
NW = NC * NS
BPW = B // NW
C = 128
NCHUNKS = BPW // C
GROUPS = C // L


def _body(head_hbm, tail_hbm, rel_hbm, ere_hbm, eim_hbm, rre_hbm, rim_hbm,
          out_hbm,
          hidx, tidx, ridx, hrow, trow, rrow, hre, him, tre, tim, rre, rim,
          outv, sem):
    wid = lax.axis_index("s") * NC + lax.axis_index("c")
    base = wid * BPW

    pltpu.sync_copy(head_hbm.at[pl.ds(base, BPW)], hidx)
    pltpu.sync_copy(tail_hbm.at[pl.ds(base, BPW)], tidx)
    pltpu.sync_copy(rel_hbm.at[pl.ds(base, BPW)], ridx)

    def rows_body(k, carry):
        sl = pl.ds(k * L, L)
        hrow[sl] = jax.lax.shift_right_logical(hidx[sl], 1)
        trow[sl] = jax.lax.shift_right_logical(tidx[sl], 1)
        rrow[sl] = jax.lax.shift_right_logical(ridx[sl], 1)
        return carry

    lax.fori_loop(0, BPW // L, rows_body, 0)

    def chunk_body(ci, carry):
        off = ci * C
        cps = [
            pltpu.async_copy(ere_hbm.at[hrow.at[pl.ds(off, C)]], hre, sem),
            pltpu.async_copy(eim_hbm.at[hrow.at[pl.ds(off, C)]], him, sem),
            pltpu.async_copy(ere_hbm.at[trow.at[pl.ds(off, C)]], tre, sem),
            pltpu.async_copy(eim_hbm.at[trow.at[pl.ds(off, C)]], tim, sem),
            pltpu.async_copy(rre_hbm.at[rrow.at[pl.ds(off, C)]], rre, sem),
            pltpu.async_copy(rim_hbm.at[rrow.at[pl.ds(off, C)]], rim, sem),
        ]
        for cp in cps:
            cp.wait()

        def grp_body(g, carry2):
            rows = g * L + lax.iota(jnp.int32, L)
            sl = pl.ds(off + g * L, L)
            hcol0 = (hidx[sl] & 1) * D
            tcol0 = (tidx[sl] & 1) * D
            rcol0 = (ridx[sl] & 1) * D

            def d_body(d, acc):
                hc = hcol0 + d
                tc = tcol0 + d
                rc = rcol0 + d
                a_hre = plsc.load_gather(hre, [rows, hc])
                a_him = plsc.load_gather(him, [rows, hc])
                a_tre = plsc.load_gather(tre, [rows, tc])
                a_tim = plsc.load_gather(tim, [rows, tc])
                a_rre = plsc.load_gather(rre, [rows, rc])
                a_rim = plsc.load_gather(rim, [rows, rc])
                sym = a_hre * a_tre + a_him * a_tim
                asym = a_hre * a_tim - a_him * a_tre
                return acc + (a_rre * sym + a_rim * asym)

            acc = lax.fori_loop(0, D, d_body, jnp.zeros((L,), jnp.float32))
            outv[pl.ds(g * L, L)] = 1.0 / (1.0 + jnp.exp(-acc))
            return carry2

        lax.fori_loop(0, GROUPS, grp_body, 0)
        pltpu.sync_copy(outv, out_hbm.at[pl.ds(base + off, C)])
        return carry

    lax.fori_loop(0, NCHUNKS, chunk_body, 0)


@jax.jit
def _run(head, tail, relation, entity_re, entity_im, relation_re, relation_im):
    f = pl.kernel(
        _body,
        out_type=jax.ShapeDtypeStruct((B,), jnp.float32),
        mesh=plsc.VectorSubcoreMesh(core_axis_name="c", subcore_axis_name="s"),
        compiler_params=pltpu.CompilerParams(needs_layout_passes=False),
        scratch_types=[
            pltpu.VMEM((BPW,), jnp.int32),
            pltpu.VMEM((BPW,), jnp.int32),
            pltpu.VMEM((BPW,), jnp.int32),
            pltpu.VMEM((BPW,), jnp.int32),
            pltpu.VMEM((BPW,), jnp.int32),
            pltpu.VMEM((BPW,), jnp.int32),
            pltpu.VMEM((C, RW), jnp.float32),
            pltpu.VMEM((C, RW), jnp.float32),
            pltpu.VMEM((C, RW), jnp.float32),
            pltpu.VMEM((C, RW), jnp.float32),
            pltpu.VMEM((C, RW), jnp.float32),
            pltpu.VMEM((C, RW), jnp.float32),
            pltpu.VMEM((C,), jnp.float32),
            pltpu.SemaphoreType.DMA,
        ],
    )
    ere2 = entity_re.reshape(E // 2, RW)
    eim2 = entity_im.reshape(E // 2, RW)
    rre2 = relation_re.reshape(R // 2, RW)
    rim2 = relation_im.reshape(R // 2, RW)
    return f(head, tail, relation, ere2, eim2, rre2, rim2)


def kernel(head, tail, relation, entity_re, entity_im, relation_re,
           relation_im):
    return _run(head, tail, relation, entity_re, entity_im, relation_re,
                relation_im)


# speedup vs baseline: 2.0631x; 2.0631x over previous
import functools

import jax
import jax.numpy as jnp
from jax import lax
from jax.experimental import pallas as pl
from jax.experimental.pallas import tpu as pltpu
from jax.experimental.pallas import tpu_sc as plsc

E = 1000000
R = 1000
B = 16384
D = 64
RW = 128
NC = 2
NS = 16
L = 16
NW = NC * NS
BPW = B // NW
C = 128
NCHUNKS = BPW // C
GROUPS = C // L
NBUF = 4


def _body(head_hbm, tail_hbm, rel_hbm, ere_hbm, eim_hbm, rre_hbm, rim_hbm,
          out_hbm,
          hidx, tidx, ridx, rrow,
          hreb, himb, treb, timb, rreb, rimb, outv, rsem, esem):
    wid = lax.axis_index("s") * NC + lax.axis_index("c")
    base = wid * BPW

    pltpu.sync_copy(head_hbm.at[pl.ds(base, BPW)], hidx)
    pltpu.sync_copy(tail_hbm.at[pl.ds(base, BPW)], tidx)
    pltpu.sync_copy(rel_hbm.at[pl.ds(base, BPW)], ridx)

    def rows_body(k, carry):
        sl = pl.ds(k * L, L)
        rrow[sl] = jax.lax.shift_right_logical(ridx[sl], 1)
        return carry

    lax.fori_loop(0, BPW // L, rows_body, 0)

    def fire_entity(h, t, slot):
        hj = jax.lax.shift_right_logical(h, 3)
        tj = jax.lax.shift_right_logical(t, 3)
        return [
            pltpu.async_copy(ere_hbm.at[hj], hreb.at[slot], esem),
            pltpu.async_copy(eim_hbm.at[hj], himb.at[slot], esem),
            pltpu.async_copy(ere_hbm.at[tj], treb.at[slot], esem),
            pltpu.async_copy(eim_hbm.at[tj], timb.at[slot], esem),
        ]

    def chunk_body(ci, carry):
        off = ci * C
        rcps = [
            pltpu.async_copy(rre_hbm.at[rrow.at[pl.ds(off, C)]], rreb, rsem),
            pltpu.async_copy(rim_hbm.at[rrow.at[pl.ds(off, C)]], rimb, rsem),
        ]
        for cp in rcps:
            cp.wait()

        def grp_body(g, carry2):
            lane_iota = lax.iota(jnp.int32, L)
            goff = g * L
            hv = hidx[pl.ds(off + goff, L)]
            tv = tidx[pl.ds(off + goff, L)]
            rv = ridx[pl.ds(off + goff, L)]

            hs = [hv[l] for l in range(L)]
            ts = [tv[l] for l in range(L)]
            rs = [rv[l] for l in range(L)]

            pend = {}
            for l in range(NBUF):
                pend[l] = fire_entity(hs[l], ts[l], l)

            scores = jnp.zeros((L,), jnp.float32)
            for l in range(L):
                for cp in pend.pop(l):
                    cp.wait()
                slot = l % NBUF

                hsub = hs[l] & 7
                tsub = ts[l] & 7
                rcol0 = (rs[l] & 1) * D
                i = goff + l

                acc = jnp.zeros((L,), jnp.float32)
                for k in range(4):
                    sl = pl.ds(k * L, L)
                    a_hre = hreb[slot, hsub, sl]
                    a_him = himb[slot, hsub, sl]
                    a_tre = treb[slot, tsub, sl]
                    a_tim = timb[slot, tsub, sl]
                    a_rre = rreb[i, pl.ds(rcol0 + k * L, L)]
                    a_rim = rimb[i, pl.ds(rcol0 + k * L, L)]
                    sym = a_hre * a_tre + a_him * a_tim
                    asym = a_hre * a_tim - a_him * a_tre
                    acc = acc + (a_rre * sym + a_rim * asym)

                s = jnp.sum(acc)
                scores = jnp.where(lane_iota == l, s, scores)

                nxt = l + NBUF
                if nxt < L:
                    pend[nxt] = fire_entity(hs[nxt], ts[nxt], nxt % NBUF)

            outv[pl.ds(goff, L)] = 1.0 / (1.0 + jnp.exp(-scores))
            return carry2

        lax.fori_loop(0, GROUPS, grp_body, 0)
        pltpu.sync_copy(outv, out_hbm.at[pl.ds(base + off, C)])
        return carry

    lax.fori_loop(0, NCHUNKS, chunk_body, 0)


@jax.jit
def _run(head, tail, relation, entity_re, entity_im, relation_re, relation_im):
    f = pl.kernel(
        _body,
        out_type=jax.ShapeDtypeStruct((B,), jnp.float32),
        mesh=plsc.VectorSubcoreMesh(core_axis_name="c", subcore_axis_name="s"),
        compiler_params=pltpu.CompilerParams(needs_layout_passes=False),
        scratch_types=[
            pltpu.VMEM((BPW,), jnp.int32),
            pltpu.VMEM((BPW,), jnp.int32),
            pltpu.VMEM((BPW,), jnp.int32),
            pltpu.VMEM((BPW,), jnp.int32),
            pltpu.VMEM((NBUF, 8, D), jnp.float32),
            pltpu.VMEM((NBUF, 8, D), jnp.float32),
            pltpu.VMEM((NBUF, 8, D), jnp.float32),
            pltpu.VMEM((NBUF, 8, D), jnp.float32),
            pltpu.VMEM((C, RW), jnp.float32),
            pltpu.VMEM((C, RW), jnp.float32),
            pltpu.VMEM((C,), jnp.float32),
            pltpu.SemaphoreType.DMA,
            pltpu.SemaphoreType.DMA,
        ],
    )
    ere3 = entity_re.reshape(E // 8, 8, D)
    eim3 = entity_im.reshape(E // 8, 8, D)
    rre2 = relation_re.reshape(R // 2, RW)
    rim2 = relation_im.reshape(R // 2, RW)
    return f(head, tail, relation, ere3, eim3, rre2, rim2)


def kernel(head, tail, relation, entity_re, entity_im, relation_re,
           relation_im):
    return _run(head, tail, relation, entity_re, entity_im, relation_re,
                relation_im)


# speedup vs baseline: 2.1637x; 1.0487x over previous
import functools

import jax
import jax.numpy as jnp
from jax import lax
from jax.experimental import pallas as pl
from jax.experimental.pallas import tpu as pltpu
from jax.experimental.pallas import tpu_sc as plsc

E = 1000000
R = 1000
B = 16384
D = 64
RW = 128
NC = 2
NS = 16
L = 16
NW = NC * NS
BPW = B // NW
C = 128
NCHUNKS = BPW // C
GROUPS = C // L
NBUF = 8


def _body(head_hbm, tail_hbm, rel_hbm, ere_hbm, eim_hbm, rre_hbm, rim_hbm,
          out_hbm,
          hidx, tidx, ridx, rrow,
          hreb, himb, treb, timb, rreb, rimb, outv, rsem, esem):
    wid = lax.axis_index("s") * NC + lax.axis_index("c")
    base = wid * BPW

    pltpu.sync_copy(head_hbm.at[pl.ds(base, BPW)], hidx)
    pltpu.sync_copy(tail_hbm.at[pl.ds(base, BPW)], tidx)
    pltpu.sync_copy(rel_hbm.at[pl.ds(base, BPW)], ridx)

    def rows_body(k, carry):
        sl = pl.ds(k * L, L)
        rrow[sl] = jax.lax.shift_right_logical(ridx[sl], 1)
        return carry

    lax.fori_loop(0, BPW // L, rows_body, 0)

    def fire_entity(h, t, slot):
        hj = jax.lax.shift_right_logical(h, 3)
        tj = jax.lax.shift_right_logical(t, 3)
        return [
            pltpu.async_copy(ere_hbm.at[hj], hreb.at[slot], esem),
            pltpu.async_copy(eim_hbm.at[hj], himb.at[slot], esem),
            pltpu.async_copy(ere_hbm.at[tj], treb.at[slot], esem),
            pltpu.async_copy(eim_hbm.at[tj], timb.at[slot], esem),
        ]

    def chunk_body(ci, carry):
        off = ci * C
        rcps = [
            pltpu.async_copy(rre_hbm.at[rrow.at[pl.ds(off, C)]], rreb, rsem),
            pltpu.async_copy(rim_hbm.at[rrow.at[pl.ds(off, C)]], rimb, rsem),
        ]
        for cp in rcps:
            cp.wait()

        def grp_body(g, carry2):
            lane_iota = lax.iota(jnp.int32, L)
            goff = g * L
            hv = hidx[pl.ds(off + goff, L)]
            tv = tidx[pl.ds(off + goff, L)]
            rv = ridx[pl.ds(off + goff, L)]

            hs = [hv[l] for l in range(L)]
            ts = [tv[l] for l in range(L)]
            rs = [rv[l] for l in range(L)]

            pend = {}
            for l in range(NBUF):
                pend[l] = fire_entity(hs[l], ts[l], l)

            scores = jnp.zeros((L,), jnp.float32)
            for l in range(L):
                for cp in pend.pop(l):
                    cp.wait()
                slot = l % NBUF

                hsub = hs[l] & 7
                tsub = ts[l] & 7
                rcol0 = (rs[l] & 1) * D
                i = goff + l

                acc = jnp.zeros((L,), jnp.float32)
                for k in range(4):
                    sl = pl.ds(k * L, L)
                    a_hre = hreb[slot, hsub, sl]
                    a_him = himb[slot, hsub, sl]
                    a_tre = treb[slot, tsub, sl]
                    a_tim = timb[slot, tsub, sl]
                    a_rre = rreb[i, pl.ds(rcol0 + k * L, L)]
                    a_rim = rimb[i, pl.ds(rcol0 + k * L, L)]
                    sym = a_hre * a_tre + a_him * a_tim
                    asym = a_hre * a_tim - a_him * a_tre
                    acc = acc + (a_rre * sym + a_rim * asym)

                s = jnp.sum(acc)
                scores = jnp.where(lane_iota == l, s, scores)

                nxt = l + NBUF
                if nxt < L:
                    pend[nxt] = fire_entity(hs[nxt], ts[nxt], nxt % NBUF)

            outv[pl.ds(goff, L)] = 1.0 / (1.0 + jnp.exp(-scores))
            return carry2

        lax.fori_loop(0, GROUPS, grp_body, 0)
        pltpu.sync_copy(outv, out_hbm.at[pl.ds(base + off, C)])
        return carry

    lax.fori_loop(0, NCHUNKS, chunk_body, 0)


@jax.jit
def _run(head, tail, relation, entity_re, entity_im, relation_re, relation_im):
    f = pl.kernel(
        _body,
        out_type=jax.ShapeDtypeStruct((B,), jnp.float32),
        mesh=plsc.VectorSubcoreMesh(core_axis_name="c", subcore_axis_name="s"),
        compiler_params=pltpu.CompilerParams(needs_layout_passes=False),
        scratch_types=[
            pltpu.VMEM((BPW,), jnp.int32),
            pltpu.VMEM((BPW,), jnp.int32),
            pltpu.VMEM((BPW,), jnp.int32),
            pltpu.VMEM((BPW,), jnp.int32),
            pltpu.VMEM((NBUF, 8, D), jnp.float32),
            pltpu.VMEM((NBUF, 8, D), jnp.float32),
            pltpu.VMEM((NBUF, 8, D), jnp.float32),
            pltpu.VMEM((NBUF, 8, D), jnp.float32),
            pltpu.VMEM((C, RW), jnp.float32),
            pltpu.VMEM((C, RW), jnp.float32),
            pltpu.VMEM((C,), jnp.float32),
            pltpu.SemaphoreType.DMA,
            pltpu.SemaphoreType.DMA,
        ],
    )
    ere3 = entity_re.reshape(E // 8, 8, D)
    eim3 = entity_im.reshape(E // 8, 8, D)
    rre2 = relation_re.reshape(R // 2, RW)
    rim2 = relation_im.reshape(R // 2, RW)
    return f(head, tail, relation, ere3, eim3, rre2, rim2)


def kernel(head, tail, relation, entity_re, entity_im, relation_re,
           relation_im):
    return _run(head, tail, relation, entity_re, entity_im, relation_re,
                relation_im)


# speedup vs baseline: 2.1722x; 1.0039x over previous
import functools

import jax
import jax.numpy as jnp
from jax import lax
from jax.experimental import pallas as pl
from jax.experimental.pallas import tpu as pltpu
from jax.experimental.pallas import tpu_sc as plsc

E = 1000000
R = 1000
B = 16384
D = 64
RW = 128
NC = 2
NS = 16
L = 16
NW = NC * NS
BPW = B // NW
C = 128
NCHUNKS = BPW // C
GROUPS = C // L
NBUF = 16


def _body(head_hbm, tail_hbm, rel_hbm, ere_hbm, eim_hbm, rre_hbm, rim_hbm,
          out_hbm,
          hidx, tidx, ridx, rrow,
          hreb, himb, treb, timb, rreb, rimb, outv, rsem, esem):
    wid = lax.axis_index("s") * NC + lax.axis_index("c")
    base = wid * BPW

    pltpu.sync_copy(head_hbm.at[pl.ds(base, BPW)], hidx)
    pltpu.sync_copy(tail_hbm.at[pl.ds(base, BPW)], tidx)
    pltpu.sync_copy(rel_hbm.at[pl.ds(base, BPW)], ridx)

    def rows_body(k, carry):
        sl = pl.ds(k * L, L)
        rrow[sl] = jax.lax.shift_right_logical(ridx[sl], 1)
        return carry

    lax.fori_loop(0, BPW // L, rows_body, 0)

    def fire_entity(h, t, slot):
        hj = jax.lax.shift_right_logical(h, 3)
        tj = jax.lax.shift_right_logical(t, 3)
        return [
            pltpu.async_copy(ere_hbm.at[hj], hreb.at[slot], esem),
            pltpu.async_copy(eim_hbm.at[hj], himb.at[slot], esem),
            pltpu.async_copy(ere_hbm.at[tj], treb.at[slot], esem),
            pltpu.async_copy(eim_hbm.at[tj], timb.at[slot], esem),
        ]

    def chunk_body(ci, carry):
        off = ci * C
        rcps = [
            pltpu.async_copy(rre_hbm.at[rrow.at[pl.ds(off, C)]], rreb, rsem),
            pltpu.async_copy(rim_hbm.at[rrow.at[pl.ds(off, C)]], rimb, rsem),
        ]
        for cp in rcps:
            cp.wait()

        def grp_body(g, carry2):
            lane_iota = lax.iota(jnp.int32, L)
            goff = g * L
            hv = hidx[pl.ds(off + goff, L)]
            tv = tidx[pl.ds(off + goff, L)]
            rv = ridx[pl.ds(off + goff, L)]

            hs = [hv[l] for l in range(L)]
            ts = [tv[l] for l in range(L)]
            rs = [rv[l] for l in range(L)]

            pend = {}
            for l in range(NBUF):
                pend[l] = fire_entity(hs[l], ts[l], l)

            scores = jnp.zeros((L,), jnp.float32)
            for l in range(L):
                for cp in pend.pop(l):
                    cp.wait()
                slot = l % NBUF

                hsub = hs[l] & 7
                tsub = ts[l] & 7
                rcol0 = (rs[l] & 1) * D
                i = goff + l

                acc = jnp.zeros((L,), jnp.float32)
                for k in range(4):
                    sl = pl.ds(k * L, L)
                    a_hre = hreb[slot, hsub, sl]
                    a_him = himb[slot, hsub, sl]
                    a_tre = treb[slot, tsub, sl]
                    a_tim = timb[slot, tsub, sl]
                    a_rre = rreb[i, pl.ds(rcol0 + k * L, L)]
                    a_rim = rimb[i, pl.ds(rcol0 + k * L, L)]
                    sym = a_hre * a_tre + a_him * a_tim
                    asym = a_hre * a_tim - a_him * a_tre
                    acc = acc + (a_rre * sym + a_rim * asym)

                s = jnp.sum(acc)
                scores = jnp.where(lane_iota == l, s, scores)

                nxt = l + NBUF
                if nxt < L:
                    pend[nxt] = fire_entity(hs[nxt], ts[nxt], nxt % NBUF)

            outv[pl.ds(goff, L)] = 1.0 / (1.0 + jnp.exp(-scores))
            return carry2

        lax.fori_loop(0, GROUPS, grp_body, 0)
        pltpu.sync_copy(outv, out_hbm.at[pl.ds(base + off, C)])
        return carry

    lax.fori_loop(0, NCHUNKS, chunk_body, 0)


@jax.jit
def _run(head, tail, relation, entity_re, entity_im, relation_re, relation_im):
    f = pl.kernel(
        _body,
        out_type=jax.ShapeDtypeStruct((B,), jnp.float32),
        mesh=plsc.VectorSubcoreMesh(core_axis_name="c", subcore_axis_name="s"),
        compiler_params=pltpu.CompilerParams(needs_layout_passes=False),
        scratch_types=[
            pltpu.VMEM((BPW,), jnp.int32),
            pltpu.VMEM((BPW,), jnp.int32),
            pltpu.VMEM((BPW,), jnp.int32),
            pltpu.VMEM((BPW,), jnp.int32),
            pltpu.VMEM((NBUF, 8, D), jnp.float32),
            pltpu.VMEM((NBUF, 8, D), jnp.float32),
            pltpu.VMEM((NBUF, 8, D), jnp.float32),
            pltpu.VMEM((NBUF, 8, D), jnp.float32),
            pltpu.VMEM((C, RW), jnp.float32),
            pltpu.VMEM((C, RW), jnp.float32),
            pltpu.VMEM((C,), jnp.float32),
            pltpu.SemaphoreType.DMA,
            pltpu.SemaphoreType.DMA,
        ],
    )
    ere3 = entity_re.reshape(E // 8, 8, D)
    eim3 = entity_im.reshape(E // 8, 8, D)
    rre2 = relation_re.reshape(R // 2, RW)
    rim2 = relation_im.reshape(R // 2, RW)
    return f(head, tail, relation, ere3, eim3, rre2, rim2)


def kernel(head, tail, relation, entity_re, entity_im, relation_re,
           relation_im):
    return _run(head, tail, relation, entity_re, entity_im, relation_re,
                relation_im)


# speedup vs baseline: 2.1766x; 1.0020x over previous
import functools

import jax
import jax.numpy as jnp
from jax import lax
from jax.experimental import pallas as pl
from jax.experimental.pallas import tpu as pltpu
from jax.experimental.pallas import tpu_sc as plsc

E = 1000000
R = 1000
B = 16384
D = 64
RW = 128
NC = 2
NS = 16
L = 16
NW = NC * NS
BPW = B // NW
C = 32
NCHUNKS = BPW // C
NBUF = 16


def _body(head_hbm, tail_hbm, rel_hbm, ere_hbm, eim_hbm, rre_hbm, rim_hbm,
          out_hbm,
          hidx, tidx, ridx, rrow,
          hreb, himb, treb, timb, rreb, rimb, outv, rsem, esem):
    wid = lax.axis_index("s") * NC + lax.axis_index("c")
    base = wid * BPW

    pltpu.sync_copy(head_hbm.at[pl.ds(base, BPW)], hidx)
    pltpu.sync_copy(tail_hbm.at[pl.ds(base, BPW)], tidx)
    pltpu.sync_copy(rel_hbm.at[pl.ds(base, BPW)], ridx)

    def rows_body(k, carry):
        sl = pl.ds(k * L, L)
        rrow[sl] = jax.lax.shift_right_logical(ridx[sl], 1)
        return carry

    lax.fori_loop(0, BPW // L, rows_body, 0)

    def fire_entity(h, t, slot):
        hj = jax.lax.shift_right_logical(h, 3)
        tj = jax.lax.shift_right_logical(t, 3)
        return [
            pltpu.async_copy(ere_hbm.at[hj], hreb.at[slot], esem),
            pltpu.async_copy(eim_hbm.at[hj], himb.at[slot], esem),
            pltpu.async_copy(ere_hbm.at[tj], treb.at[slot], esem),
            pltpu.async_copy(eim_hbm.at[tj], timb.at[slot], esem),
        ]

    def chunk_body(ci, carry):
        off = ci * C
        lane_iota = lax.iota(jnp.int32, L)
        rcps = [
            pltpu.async_copy(rre_hbm.at[rrow.at[pl.ds(off, C)]], rreb, rsem),
            pltpu.async_copy(rim_hbm.at[rrow.at[pl.ds(off, C)]], rimb, rsem),
        ]
        for cp in rcps:
            cp.wait()

        hs, ts, rs = [], [], []
        for g in range(C // L):
            hv = hidx[pl.ds(off + g * L, L)]
            tv = tidx[pl.ds(off + g * L, L)]
            rv = ridx[pl.ds(off + g * L, L)]
            hs += [hv[l] for l in range(L)]
            ts += [tv[l] for l in range(L)]
            rs += [rv[l] for l in range(L)]

        pend = {}
        for l in range(NBUF):
            pend[l] = fire_entity(hs[l], ts[l], l)

        scores = jnp.zeros((L,), jnp.float32)
        for l in range(C):
            for cp in pend.pop(l):
                cp.wait()
            slot = l % NBUF

            hsub = hs[l] & 7
            tsub = ts[l] & 7
            rcol0 = (rs[l] & 1) * D

            acc = jnp.zeros((L,), jnp.float32)
            for k in range(4):
                sl = pl.ds(k * L, L)
                a_hre = hreb[slot, hsub, sl]
                a_him = himb[slot, hsub, sl]
                a_tre = treb[slot, tsub, sl]
                a_tim = timb[slot, tsub, sl]
                a_rre = rreb[l, pl.ds(rcol0 + k * L, L)]
                a_rim = rimb[l, pl.ds(rcol0 + k * L, L)]
                sym = a_hre * a_tre + a_him * a_tim
                asym = a_hre * a_tim - a_him * a_tre
                acc = acc + (a_rre * sym + a_rim * asym)

            s = jnp.sum(acc)
            scores = jnp.where(lane_iota == (l % L), s, scores)

            if l % L == L - 1:
                outv[pl.ds((l // L) * L, L)] = 1.0 / (1.0 + jnp.exp(-scores))
                scores = jnp.zeros((L,), jnp.float32)

            nxt = l + NBUF
            if nxt < C:
                pend[nxt] = fire_entity(hs[nxt], ts[nxt], nxt % NBUF)

        pltpu.sync_copy(outv, out_hbm.at[pl.ds(base + off, C)])
        return carry

    lax.fori_loop(0, NCHUNKS, chunk_body, 0)


@jax.jit
def _run(head, tail, relation, entity_re, entity_im, relation_re, relation_im):
    f = pl.kernel(
        _body,
        out_type=jax.ShapeDtypeStruct((B,), jnp.float32),
        mesh=plsc.VectorSubcoreMesh(core_axis_name="c", subcore_axis_name="s"),
        compiler_params=pltpu.CompilerParams(needs_layout_passes=False),
        scratch_types=[
            pltpu.VMEM((BPW,), jnp.int32),
            pltpu.VMEM((BPW,), jnp.int32),
            pltpu.VMEM((BPW,), jnp.int32),
            pltpu.VMEM((BPW,), jnp.int32),
            pltpu.VMEM((NBUF, 8, D), jnp.float32),
            pltpu.VMEM((NBUF, 8, D), jnp.float32),
            pltpu.VMEM((NBUF, 8, D), jnp.float32),
            pltpu.VMEM((NBUF, 8, D), jnp.float32),
            pltpu.VMEM((C, RW), jnp.float32),
            pltpu.VMEM((C, RW), jnp.float32),
            pltpu.VMEM((C,), jnp.float32),
            pltpu.SemaphoreType.DMA,
            pltpu.SemaphoreType.DMA,
        ],
    )
    ere3 = entity_re.reshape(E // 8, 8, D)
    eim3 = entity_im.reshape(E // 8, 8, D)
    rre2 = relation_re.reshape(R // 2, RW)
    rim2 = relation_im.reshape(R // 2, RW)
    return f(head, tail, relation, ere3, eim3, rre2, rim2)


def kernel(head, tail, relation, entity_re, entity_im, relation_re,
           relation_im):
    return _run(head, tail, relation, entity_re, entity_im, relation_re,
                relation_im)


# speedup vs baseline: 2.1827x; 1.0028x over previous
import functools

import jax
import jax.numpy as jnp
from jax import lax
from jax.experimental import pallas as pl
from jax.experimental.pallas import tpu as pltpu
from jax.experimental.pallas import tpu_sc as plsc

E = 1000000
R = 1000
B = 16384
D = 64
RW = 128
NC = 2
NS = 16
L = 16
NW = NC * NS
BPW = B // NW
C = 64
NCHUNKS = BPW // C
NBUF = 16


def _body(head_hbm, tail_hbm, rel_hbm, ere_hbm, eim_hbm, rre_hbm, rim_hbm,
          out_hbm,
          hidx, tidx, ridx, rrow,
          hreb, himb, treb, timb, rreb, rimb, outv, rsem, esem):
    wid = lax.axis_index("s") * NC + lax.axis_index("c")
    base = wid * BPW

    pltpu.sync_copy(head_hbm.at[pl.ds(base, BPW)], hidx)
    pltpu.sync_copy(tail_hbm.at[pl.ds(base, BPW)], tidx)
    pltpu.sync_copy(rel_hbm.at[pl.ds(base, BPW)], ridx)

    def rows_body(k, carry):
        sl = pl.ds(k * L, L)
        rrow[sl] = jax.lax.shift_right_logical(ridx[sl], 1)
        return carry

    lax.fori_loop(0, BPW // L, rows_body, 0)

    def fire_entity(h, t, slot):
        hj = jax.lax.shift_right_logical(h, 3)
        tj = jax.lax.shift_right_logical(t, 3)
        return [
            pltpu.async_copy(ere_hbm.at[hj], hreb.at[slot], esem),
            pltpu.async_copy(eim_hbm.at[hj], himb.at[slot], esem),
            pltpu.async_copy(ere_hbm.at[tj], treb.at[slot], esem),
            pltpu.async_copy(eim_hbm.at[tj], timb.at[slot], esem),
        ]

    def chunk_body(ci, carry):
        off = ci * C
        lane_iota = lax.iota(jnp.int32, L)
        rcps = [
            pltpu.async_copy(rre_hbm.at[rrow.at[pl.ds(off, C)]], rreb, rsem),
            pltpu.async_copy(rim_hbm.at[rrow.at[pl.ds(off, C)]], rimb, rsem),
        ]
        for cp in rcps:
            cp.wait()

        hs, ts, rs = [], [], []
        for g in range(C // L):
            hv = hidx[pl.ds(off + g * L, L)]
            tv = tidx[pl.ds(off + g * L, L)]
            rv = ridx[pl.ds(off + g * L, L)]
            hs += [hv[l] for l in range(L)]
            ts += [tv[l] for l in range(L)]
            rs += [rv[l] for l in range(L)]

        pend = {}
        for l in range(NBUF):
            pend[l] = fire_entity(hs[l], ts[l], l)

        scores = jnp.zeros((L,), jnp.float32)
        for l in range(C):
            for cp in pend.pop(l):
                cp.wait()
            slot = l % NBUF

            hsub = hs[l] & 7
            tsub = ts[l] & 7
            rcol0 = (rs[l] & 1) * D

            acc = jnp.zeros((L,), jnp.float32)
            for k in range(4):
                sl = pl.ds(k * L, L)
                a_hre = hreb[slot, hsub, sl]
                a_him = himb[slot, hsub, sl]
                a_tre = treb[slot, tsub, sl]
                a_tim = timb[slot, tsub, sl]
                a_rre = rreb[l, pl.ds(rcol0 + k * L, L)]
                a_rim = rimb[l, pl.ds(rcol0 + k * L, L)]
                sym = a_hre * a_tre + a_him * a_tim
                asym = a_hre * a_tim - a_him * a_tre
                acc = acc + (a_rre * sym + a_rim * asym)

            s = jnp.sum(acc)
            scores = jnp.where(lane_iota == (l % L), s, scores)

            if l % L == L - 1:
                outv[pl.ds((l // L) * L, L)] = 1.0 / (1.0 + jnp.exp(-scores))
                scores = jnp.zeros((L,), jnp.float32)

            nxt = l + NBUF
            if nxt < C:
                pend[nxt] = fire_entity(hs[nxt], ts[nxt], nxt % NBUF)

        pltpu.sync_copy(outv, out_hbm.at[pl.ds(base + off, C)])
        return carry

    lax.fori_loop(0, NCHUNKS, chunk_body, 0)


@jax.jit
def _run(head, tail, relation, entity_re, entity_im, relation_re, relation_im):
    f = pl.kernel(
        _body,
        out_type=jax.ShapeDtypeStruct((B,), jnp.float32),
        mesh=plsc.VectorSubcoreMesh(core_axis_name="c", subcore_axis_name="s"),
        compiler_params=pltpu.CompilerParams(needs_layout_passes=False),
        scratch_types=[
            pltpu.VMEM((BPW,), jnp.int32),
            pltpu.VMEM((BPW,), jnp.int32),
            pltpu.VMEM((BPW,), jnp.int32),
            pltpu.VMEM((BPW,), jnp.int32),
            pltpu.VMEM((NBUF, 8, D), jnp.float32),
            pltpu.VMEM((NBUF, 8, D), jnp.float32),
            pltpu.VMEM((NBUF, 8, D), jnp.float32),
            pltpu.VMEM((NBUF, 8, D), jnp.float32),
            pltpu.VMEM((C, RW), jnp.float32),
            pltpu.VMEM((C, RW), jnp.float32),
            pltpu.VMEM((C,), jnp.float32),
            pltpu.SemaphoreType.DMA,
            pltpu.SemaphoreType.DMA,
        ],
    )
    ere3 = entity_re.reshape(E // 8, 8, D)
    eim3 = entity_im.reshape(E // 8, 8, D)
    rre2 = relation_re.reshape(R // 2, RW)
    rim2 = relation_im.reshape(R // 2, RW)
    return f(head, tail, relation, ere3, eim3, rre2, rim2)


def kernel(head, tail, relation, entity_re, entity_im, relation_re,
           relation_im):
    return _run(head, tail, relation, entity_re, entity_im, relation_re,
                relation_im)
